# baseline (speedup 1.0000x reference)
import functools

import jax
import jax.numpy as jnp
from jax import lax
from jax.experimental import pallas as pl
from jax.experimental.pallas import tpu as pltpu
from jax.experimental.pallas import tpu_sc as plsc

D = 128
K = 10
KP = 16
M = 64
HE = 514
HN = 256
RB = 256
RBK = 1024
RBL = 1024
GR = 128
NW = 32
BF = jnp.bfloat16
F32 = jnp.float32


def _dot(a, b):
    return jnp.dot(a, b, preferred_element_type=F32)


def _silu(x):
    return (0.5 * x) * (jnp.tanh(0.5 * x) + 1.0)


def _knn_l1_body(xi_ref, xj_ref, e_ref, wi_ref, wj_ref, wd_ref, be1_ref,
                 we2_ref, be2_ref, wn1a_ref, wn1b_ref, bn1_ref, wn2_ref,
                 bn2_ref, idx_ref, dst_ref, o_ref):
    nloc = xj_ref.shape[2]
    xi = xi_ref[0]
    xj = xj_ref[0]
    dx = xi[:, 0:1] - xj[0:1, :]
    dy = xi[:, 1:2] - xj[1:2, :]
    dz = xi[:, 2:3] - xj[2:3, :]
    d = (dx * dx + dy * dy) + dz * dz
    cols = lax.broadcasted_iota(jnp.int32, d.shape, 1)

    e = e_ref[...]
    eb = e.astype(BF)
    c0 = _dot(eb, wi_ref[...]) + _dot(eb, wj_ref[...]) + be1_ref[...]
    wd = wd_ref[...]
    we2 = we2_ref[...]
    be2 = be2_ref[...]
    msum = jnp.zeros((RBK, M), F32)
    for t in range(K):
        m = jnp.min(d, axis=1, keepdims=True)
        idx = jnp.min(jnp.where(d == m, cols, nloc), axis=1, keepdims=True)
        idx_ref[0, :, t:t + 1] = idx
        dst_ref[0, :, t:t + 1] = m
        d = jnp.where(cols == idx, jnp.inf, d)
        h1 = _silu(c0 + m * wd)
        msum = msum + _silu(_dot(h1.astype(BF), we2) + be2)

    u = _silu(_dot(eb, wn1a_ref[...])
              + _dot(msum.astype(BF), wn1b_ref[...]) + bn1_ref[...])
    o_ref[0] = _dot(u.astype(BF), wn2_ref[...]) + bn2_ref[...] + e


def _knn_l1(x, e, w, l):
    xj = jnp.transpose(x, (0, 2, 1))
    wspecs = [pl.BlockSpec(a.shape, lambda i: (0, 0)) for a in w]
    return pl.pallas_call(
        _knn_l1_body,
        grid=(l // RBK,),
        in_specs=[
            pl.BlockSpec((1, RBK, 3), lambda i: (0, i, 0)),
            pl.BlockSpec((1, 3, l), lambda i: (0, 0, 0)),
            pl.BlockSpec((1, D), lambda i: (0, 0)),
        ] + wspecs,
        out_specs=[
            pl.BlockSpec((1, RBK, KP), lambda i: (0, i, 0)),
            pl.BlockSpec((1, RBK, KP), lambda i: (0, i, 0)),
            pl.BlockSpec((1, RBK, D), lambda i: (0, i, 0)),
        ],
        out_shape=[
            jax.ShapeDtypeStruct((1, l, KP), jnp.int32),
            jax.ShapeDtypeStruct((1, l, KP), F32),
            jax.ShapeDtypeStruct((1, l, D), F32),
        ],
    )(x, xj, e, *w)


def _gather_rows(table, idx3):
    nw, nch, _ = idx3.shape
    mesh = plsc.VectorSubcoreMesh(core_axis_name="c", subcore_axis_name="s")

    @functools.partial(
        pl.kernel,
        mesh=mesh,
        out_type=jax.ShapeDtypeStruct((nw * nch * GR, D), F32),
        scratch_types=[
            pltpu.VMEM((nch, GR), jnp.int32),
            pltpu.VMEM((GR, D), F32),
            pltpu.VMEM((GR, D), F32),
            pltpu.SemaphoreType.DMA,
            pltpu.SemaphoreType.DMA,
        ],
    )
    def run(table_hbm, idx_hbm, out_hbm, idx_v, rows0, rows1, sem0, sem1):
        wid = lax.axis_index("s") * 2 + lax.axis_index("c")
        pltpu.sync_copy(idx_hbm.at[wid], idx_v)
        bufs = (rows0, rows1)
        sems = (sem0, sem1)
        cps = [None, None]
        cps[0] = pltpu.async_copy(table_hbm.at[idx_v.at[0]], rows0, sem0)
        for ch in range(1, nch):
            cps[ch % 2] = pltpu.async_copy(
                table_hbm.at[idx_v.at[ch]], bufs[ch % 2], sems[ch % 2])
            prev = ch - 1
            cps[prev % 2].wait()
            pltpu.sync_copy(bufs[prev % 2],
                            out_hbm.at[pl.ds((wid * nch + prev) * GR, GR)])
        cps[(nch - 1) % 2].wait()
        pltpu.sync_copy(bufs[(nch - 1) % 2],
                        out_hbm.at[pl.ds((wid * nch + nch - 1) * GR, GR)])

    return run(table, idx3)


def _layer_body(f_ref, g_ref, d_ref, wi_ref, wj_ref, wd_ref, be1_ref,
                we2_ref, be2_ref, wn1a_ref, wn1b_ref, bn1_ref, wn2_ref,
                bn2_ref, *rest):
    *ln_refs, o_ref = rest
    f = f_ref[0]
    fb = f.astype(BF)
    a = _dot(fb, wi_ref[...]) + be1_ref[...]
    wj = wj_ref[...]
    wd = wd_ref[...]
    we2 = we2_ref[...]
    be2 = be2_ref[...]
    msum = jnp.zeros((RBL, M), F32)
    for t in range(K):
        pre = a + _dot(g_ref[0, t].astype(BF), wj) + d_ref[0, :, t:t + 1] * wd
        h1 = _silu(pre)
        msum = msum + _silu(_dot(h1.astype(BF), we2) + be2)

    u = _silu(_dot(fb, wn1a_ref[...])
              + _dot(msum.astype(BF), wn1b_ref[...]) + bn1_ref[...])
    out = _dot(u.astype(BF), wn2_ref[...]) + bn2_ref[...] + f
    if ln_refs:
        gamma_ref, beta_ref = ln_refs
        mu = jnp.mean(out, axis=1, keepdims=True)
        var = jnp.mean((out - mu) ** 2, axis=1, keepdims=True)
        out = ((out - mu) * lax.rsqrt(var + 1e-5) * gamma_ref[...]
               + beta_ref[...])
    o_ref[0] = out


def _prep_weights(p):
    we1 = p['W_e1']
    wn1 = p['W_n1']
    c = lambda a: a.astype(BF)
    return (c(we1[:D]), c(we1[D:2 * D]), we1[2 * D:2 * D + 1],
            p['b_e1'][None, :], c(p['W_e2']), p['b_e2'][None, :],
            c(wn1[:D]), c(wn1[D:]), p['b_n1'][None, :],
            c(p['W_n2']), p['b_n2'][None, :])


def _layer(f, g, dist_k, w, ln, l):
    wspecs = [pl.BlockSpec(a.shape, lambda i: (0, 0)) for a in w]
    extra_specs = []
    extra_args = ()
    if ln is not None:
        extra_specs = [pl.BlockSpec((1, D), lambda i: (0, 0)),
                       pl.BlockSpec((1, D), lambda i: (0, 0))]
        extra_args = (ln[0][None, :], ln[1][None, :])
    return pl.pallas_call(
        _layer_body,
        grid=(l // RBL,),
        in_specs=[pl.BlockSpec((1, RBL, D), lambda i: (0, i, 0)),
                  pl.BlockSpec((1, K, RBL, D), lambda i: (0, 0, i, 0)),
                  pl.BlockSpec((1, RBL, KP), lambda i: (0, i, 0))]
                 + wspecs + extra_specs,
        out_specs=pl.BlockSpec((1, RBL, D), lambda i: (0, i, 0)),
        out_shape=jax.ShapeDtypeStruct((1, l, D), F32),
    )(f, g, dist_k, *w, *extra_args)


def kernel(coords, params):
    b, l = coords.shape[0], coords.shape[1]
    e = params['node_embedding'].reshape(1, D)
    ws = [_prep_weights(p) for p in params['layers']]
    ln = (params['ln_gamma'], params['ln_beta'])
    nch = (K * l) // (NW * GR)

    nbhd = [None] * b
    dist_k = [None] * b
    h = [None] * b
    idx3 = [None] * b
    g = [None] * b

    def gather(bi):
        return _gather_rows(h[bi].reshape(l, D), idx3[bi]).reshape(1, K, l, D)

    for bi in range(b):
        x = coords[bi:bi + 1, :, 1, :]
        nbhd[bi], dist_k[bi], h[bi] = _knn_l1(x, e, ws[0], l)
        idx3[bi] = jnp.transpose(nbhd[bi][0, :, :K], (1, 0)).reshape(NW, nch, GR)
        g[bi] = gather(bi)
    for bi in range(b):
        h[bi] = _layer(h[bi], g[bi], dist_k[bi], ws[1], None, l)
        g[bi] = gather(bi)
    for bi in range(b):
        h[bi] = _layer(h[bi], g[bi], dist_k[bi], ws[2], ln, l)
    return jnp.concatenate(h, axis=0)


# speedup vs baseline: 1.0957x; 1.0957x over previous
import functools

import jax
import jax.numpy as jnp
from jax import lax
from jax.experimental import pallas as pl
from jax.experimental.pallas import tpu as pltpu
from jax.experimental.pallas import tpu_sc as plsc

D = 128
K = 10
KP = 16
M = 64
HE = 514
HN = 256
RB = 256
RBK = 512
RBL = 512
GR = 128
NW = 32
BF = jnp.bfloat16
F32 = jnp.float32


def _dot(a, b):
    return jnp.dot(a, b, preferred_element_type=F32)


def _silu(x):
    return (0.5 * x) * (jnp.tanh(0.5 * x) + 1.0)


def _knn_l1_body(xi_ref, xj_ref, e_ref, wi_ref, wj_ref, wd_ref, be1_ref,
                 we2_ref, be2_ref, wn1a_ref, wn1b_ref, bn1_ref, wn2_ref,
                 bn2_ref, idx_ref, dst_ref, o_ref):
    nloc = xj_ref.shape[2]
    xi = xi_ref[0]
    xj = xj_ref[0]
    dx = xi[:, 0:1] - xj[0:1, :]
    dy = xi[:, 1:2] - xj[1:2, :]
    dz = xi[:, 2:3] - xj[2:3, :]
    d = (dx * dx + dy * dy) + dz * dz
    cols = lax.broadcasted_iota(jnp.int32, d.shape, 1)

    e = e_ref[...]
    eb = e.astype(BF)
    c0 = _dot(eb, wi_ref[...]) + _dot(eb, wj_ref[...]) + be1_ref[...]
    wd = wd_ref[...]
    we2 = we2_ref[...]
    be2 = be2_ref[...]
    msum = jnp.zeros((RBK, M), F32)
    for t in range(K):
        m = jnp.min(d, axis=1, keepdims=True)
        idx = jnp.min(jnp.where(d == m, cols, nloc), axis=1, keepdims=True)
        idx_ref[0, :, t:t + 1] = idx
        dst_ref[0, :, t:t + 1] = m
        d = jnp.where(cols == idx, jnp.inf, d)
        h1 = _silu(c0 + m * wd)
        msum = msum + _silu(_dot(h1.astype(BF), we2) + be2)

    u = _silu(_dot(eb, wn1a_ref[...])
              + _dot(msum.astype(BF), wn1b_ref[...]) + bn1_ref[...])
    o_ref[0] = _dot(u.astype(BF), wn2_ref[...]) + bn2_ref[...] + e


def _knn_l1(x, e, w, l):
    xj = jnp.transpose(x, (0, 2, 1))
    wspecs = [pl.BlockSpec(a.shape, lambda i: (0, 0)) for a in w]
    return pl.pallas_call(
        _knn_l1_body,
        grid=(l // RBK,),
        in_specs=[
            pl.BlockSpec((1, RBK, 3), lambda i: (0, i, 0)),
            pl.BlockSpec((1, 3, l), lambda i: (0, 0, 0)),
            pl.BlockSpec((1, D), lambda i: (0, 0)),
        ] + wspecs,
        out_specs=[
            pl.BlockSpec((1, RBK, KP), lambda i: (0, i, 0)),
            pl.BlockSpec((1, RBK, KP), lambda i: (0, i, 0)),
            pl.BlockSpec((1, RBK, D), lambda i: (0, i, 0)),
        ],
        out_shape=[
            jax.ShapeDtypeStruct((1, l, KP), jnp.int32),
            jax.ShapeDtypeStruct((1, l, KP), F32),
            jax.ShapeDtypeStruct((1, l, D), F32),
        ],
    )(x, xj, e, *w)


def _gather_rows(table, idx3):
    nw, nch, _ = idx3.shape
    mesh = plsc.VectorSubcoreMesh(core_axis_name="c", subcore_axis_name="s")

    @functools.partial(
        pl.kernel,
        mesh=mesh,
        out_type=jax.ShapeDtypeStruct((nw * nch * GR, D), F32),
        scratch_types=[
            pltpu.VMEM((nch, GR), jnp.int32),
            pltpu.VMEM((GR, D), F32),
            pltpu.VMEM((GR, D), F32),
            pltpu.SemaphoreType.DMA,
            pltpu.SemaphoreType.DMA,
        ],
    )
    def run(table_hbm, idx_hbm, out_hbm, idx_v, rows0, rows1, sem0, sem1):
        wid = lax.axis_index("s") * 2 + lax.axis_index("c")
        pltpu.sync_copy(idx_hbm.at[wid], idx_v)
        bufs = (rows0, rows1)
        sems = (sem0, sem1)
        cps = [None, None]
        cps[0] = pltpu.async_copy(table_hbm.at[idx_v.at[0]], rows0, sem0)
        for ch in range(1, nch):
            cps[ch % 2] = pltpu.async_copy(
                table_hbm.at[idx_v.at[ch]], bufs[ch % 2], sems[ch % 2])
            prev = ch - 1
            cps[prev % 2].wait()
            pltpu.sync_copy(bufs[prev % 2],
                            out_hbm.at[pl.ds((wid * nch + prev) * GR, GR)])
        cps[(nch - 1) % 2].wait()
        pltpu.sync_copy(bufs[(nch - 1) % 2],
                        out_hbm.at[pl.ds((wid * nch + nch - 1) * GR, GR)])

    return run(table, idx3)


def _layer_body(f_ref, g_ref, d_ref, wi_ref, wj_ref, wd_ref, be1_ref,
                we2_ref, be2_ref, wn1a_ref, wn1b_ref, bn1_ref, wn2_ref,
                bn2_ref, *rest):
    *ln_refs, o_ref = rest
    f = f_ref[0]
    fb = f.astype(BF)
    a = _dot(fb, wi_ref[...]) + be1_ref[...]
    wj = wj_ref[...]
    wd = wd_ref[...]
    we2 = we2_ref[...]
    be2 = be2_ref[...]
    msum = jnp.zeros((RBL, M), F32)
    for t in range(K):
        pre = a + _dot(g_ref[0, t].astype(BF), wj) + d_ref[0, :, t:t + 1] * wd
        h1 = _silu(pre)
        msum = msum + _silu(_dot(h1.astype(BF), we2) + be2)

    u = _silu(_dot(fb, wn1a_ref[...])
              + _dot(msum.astype(BF), wn1b_ref[...]) + bn1_ref[...])
    out = _dot(u.astype(BF), wn2_ref[...]) + bn2_ref[...] + f
    if ln_refs:
        gamma_ref, beta_ref = ln_refs
        mu = jnp.mean(out, axis=1, keepdims=True)
        var = jnp.mean((out - mu) ** 2, axis=1, keepdims=True)
        out = ((out - mu) * lax.rsqrt(var + 1e-5) * gamma_ref[...]
               + beta_ref[...])
    o_ref[0] = out


def _prep_weights(p):
    we1 = p['W_e1']
    wn1 = p['W_n1']
    c = lambda a: a.astype(BF)
    return (c(we1[:D]), c(we1[D:2 * D]), we1[2 * D:2 * D + 1],
            p['b_e1'][None, :], c(p['W_e2']), p['b_e2'][None, :],
            c(wn1[:D]), c(wn1[D:]), p['b_n1'][None, :],
            c(p['W_n2']), p['b_n2'][None, :])


def _layer(f, g, dist_k, w, ln, l):
    wspecs = [pl.BlockSpec(a.shape, lambda i: (0, 0)) for a in w]
    extra_specs = []
    extra_args = ()
    if ln is not None:
        extra_specs = [pl.BlockSpec((1, D), lambda i: (0, 0)),
                       pl.BlockSpec((1, D), lambda i: (0, 0))]
        extra_args = (ln[0][None, :], ln[1][None, :])
    return pl.pallas_call(
        _layer_body,
        grid=(l // RBL,),
        in_specs=[pl.BlockSpec((1, RBL, D), lambda i: (0, i, 0)),
                  pl.BlockSpec((1, K, RBL, D), lambda i: (0, 0, i, 0)),
                  pl.BlockSpec((1, RBL, KP), lambda i: (0, i, 0))]
                 + wspecs + extra_specs,
        out_specs=pl.BlockSpec((1, RBL, D), lambda i: (0, i, 0)),
        out_shape=jax.ShapeDtypeStruct((1, l, D), F32),
    )(f, g, dist_k, *w, *extra_args)


def kernel(coords, params):
    b, l = coords.shape[0], coords.shape[1]
    e = params['node_embedding'].reshape(1, D)
    ws = [_prep_weights(p) for p in params['layers']]
    ln = (params['ln_gamma'], params['ln_beta'])
    nch = (K * l) // (NW * GR)

    nbhd = [None] * b
    dist_k = [None] * b
    h = [None] * b
    idx3 = [None] * b
    g = [None] * b

    def gather(bi):
        return _gather_rows(h[bi].reshape(l, D), idx3[bi]).reshape(1, K, l, D)

    for bi in range(b):
        x = coords[bi:bi + 1, :, 1, :]
        nbhd[bi], dist_k[bi], h[bi] = _knn_l1(x, e, ws[0], l)
        idx3[bi] = jnp.transpose(nbhd[bi][0, :, :K], (1, 0)).reshape(NW, nch, GR)
        g[bi] = gather(bi)
    for bi in range(b):
        h[bi] = _layer(h[bi], g[bi], dist_k[bi], ws[1], None, l)
        g[bi] = gather(bi)
    for bi in range(b):
        h[bi] = _layer(h[bi], g[bi], dist_k[bi], ws[2], ln, l)
    return jnp.concatenate(h, axis=0)
